# Initial kernel scaffold; baseline (speedup 1.0000x reference)
import jax
import jax.numpy as jnp
from jax.experimental import pallas as pl


def kernel(inputs, W):
    raise NotImplementedError("write your pallas kernel here")


# speedup vs baseline: 4.0589x; 4.0589x over previous
import functools

import jax
import jax.numpy as jnp
from jax.experimental import pallas as pl
from jax.experimental.pallas import tpu as pltpu

NUM_EMBEDDINGS = 1024
CODE_DIM = 32
COMMITMENT_COST = 0.25

BLOCK_N = 4096


def _vq_body(x_ref, xsq_ref, wt_ref, w_ref, wsq_ref, q_ref, loss_ref, perp_ref,
             acc_ref, cnt_ref, *, n_total, n_blocks):
    step = pl.program_id(0)

    x = x_ref[...]
    wt = wt_ref[...]
    scores = jnp.dot(x, wt, preferred_element_type=jnp.float32)
    dist = (xsq_ref[...] + wsq_ref[...]) - 2.0 * scores
    dmin = jnp.min(dist, axis=1, keepdims=True)
    col = jax.lax.broadcasted_iota(jnp.int32, dist.shape, 1)
    idx = jnp.min(jnp.where(dist == dmin, col, NUM_EMBEDDINGS),
                  axis=1, keepdims=True)

    onehot = (col == idx).astype(jnp.float32)
    q = jnp.dot(onehot, w_ref[...], preferred_element_type=jnp.float32)
    q_ref[...] = q

    diff = q - x
    blk_sq = jnp.sum(diff * diff)
    blk_cnt = jnp.sum(onehot, axis=0, keepdims=True)

    @pl.when(step == 0)
    def _init():
        acc_ref[0, 0] = blk_sq
        cnt_ref[...] = blk_cnt

    @pl.when(step > 0)
    def _acc():
        acc_ref[0, 0] += blk_sq
        cnt_ref[...] += blk_cnt

    @pl.when(step == n_blocks - 1)
    def _finalize():
        total_sq = acc_ref[0, 0]
        loss = (1.0 + COMMITMENT_COST) * total_sq / (n_total * CODE_DIM)
        loss_ref[...] = jnp.reshape(loss, (1, 1))
        probs = cnt_ref[...] / n_total
        ent = jnp.sum(probs * jnp.log(probs + 1e-10), axis=1, keepdims=True)
        perp_ref[...] = jnp.exp(-ent)


def kernel(inputs, W):
    input_shape = inputs.shape
    flat = inputs.reshape(-1, CODE_DIM)
    n_total = flat.shape[0]
    n_blocks = n_total // BLOCK_N

    wt = W.T
    wsq = jnp.sum(W ** 2, axis=1)[None, :]
    xsq = jnp.sum(flat ** 2, axis=1, keepdims=True)

    grid = (n_blocks,)
    q, loss, perp = pl.pallas_call(
        functools.partial(_vq_body, n_total=n_total, n_blocks=n_blocks),
        grid=grid,
        in_specs=[
            pl.BlockSpec((BLOCK_N, CODE_DIM), lambda i: (i, 0)),
            pl.BlockSpec((BLOCK_N, 1), lambda i: (i, 0)),
            pl.BlockSpec((CODE_DIM, NUM_EMBEDDINGS), lambda i: (0, 0)),
            pl.BlockSpec((NUM_EMBEDDINGS, CODE_DIM), lambda i: (0, 0)),
            pl.BlockSpec((1, NUM_EMBEDDINGS), lambda i: (0, 0)),
        ],
        out_specs=[
            pl.BlockSpec((BLOCK_N, CODE_DIM), lambda i: (i, 0)),
            pl.BlockSpec((1, 1), lambda i: (0, 0)),
            pl.BlockSpec((1, 1), lambda i: (0, 0)),
        ],
        out_shape=[
            jax.ShapeDtypeStruct((n_total, CODE_DIM), jnp.float32),
            jax.ShapeDtypeStruct((1, 1), jnp.float32),
            jax.ShapeDtypeStruct((1, 1), jnp.float32),
        ],
        scratch_shapes=[
            pltpu.SMEM((1, 1), jnp.float32),
            pltpu.VMEM((1, NUM_EMBEDDINGS), jnp.float32),
        ],
    )(flat, xsq, wt, W, wsq)

    return (q.reshape(input_shape), loss[0, 0], perp[0, 0])


# speedup vs baseline: 5.2314x; 1.2889x over previous
import functools

import jax
import jax.numpy as jnp
from jax.experimental import pallas as pl
from jax.experimental.pallas import tpu as pltpu

NUM_EMBEDDINGS = 1024
EMBEDDING_DIM = 256
SAMPLE_TOKENS = 8
CODE_DIM = 32
COMMITMENT_COST = 0.25

BLOCK_T = 512


def _vq_body(x_ref, wt_ref, w_ref, wsq_ref, q_ref, loss_ref, perp_ref,
             acc_ref, cnt_ref, *, n_rows, n_blocks):
    step = pl.program_id(0)

    wt = wt_ref[...]
    w = w_ref[...]
    wsq = wsq_ref[...]

    blk_sq = None
    blk_cnt = None
    for s in range(SAMPLE_TOKENS):
        xs = x_ref[:, s * CODE_DIM:(s + 1) * CODE_DIM]
        scores = jnp.dot(xs, wt, preferred_element_type=jnp.float32)
        xsq = jnp.sum(xs * xs, axis=1, keepdims=True)
        dist = (xsq + wsq) - 2.0 * scores
        dmin = jnp.min(dist, axis=1, keepdims=True)
        col = jax.lax.broadcasted_iota(jnp.int32, dist.shape, 1)
        idx = jnp.min(jnp.where(dist == dmin, col, NUM_EMBEDDINGS),
                      axis=1, keepdims=True)
        onehot = (col == idx).astype(jnp.float32)
        q = jnp.dot(onehot, w, preferred_element_type=jnp.float32)
        q_ref[:, s * CODE_DIM:(s + 1) * CODE_DIM] = q

        diff = q - xs
        sq_s = jnp.sum(diff * diff)
        cnt_s = jnp.sum(onehot, axis=0, keepdims=True)
        blk_sq = sq_s if blk_sq is None else blk_sq + sq_s
        blk_cnt = cnt_s if blk_cnt is None else blk_cnt + cnt_s

    @pl.when(step == 0)
    def _init():
        acc_ref[0, 0] = blk_sq
        cnt_ref[...] = blk_cnt

    @pl.when(step > 0)
    def _acc():
        acc_ref[0, 0] += blk_sq
        cnt_ref[...] += blk_cnt

    @pl.when(step == n_blocks - 1)
    def _finalize():
        total_sq = acc_ref[0, 0]
        loss = (1.0 + COMMITMENT_COST) * total_sq / (n_rows * CODE_DIM)
        loss_ref[...] = jnp.reshape(loss, (1, 1))
        probs = cnt_ref[...] / n_rows
        ent = jnp.sum(probs * jnp.log(probs + 1e-10), axis=1, keepdims=True)
        perp_ref[...] = jnp.exp(-ent)


def kernel(inputs, W):
    input_shape = inputs.shape
    x2 = inputs.reshape(-1, EMBEDDING_DIM)
    n_tok = x2.shape[0]
    n_rows = n_tok * SAMPLE_TOKENS
    n_blocks = n_tok // BLOCK_T

    wt = W.T
    wsq = jnp.sum(W ** 2, axis=1)[None, :]

    q, loss, perp = pl.pallas_call(
        functools.partial(_vq_body, n_rows=n_rows, n_blocks=n_blocks),
        grid=(n_blocks,),
        in_specs=[
            pl.BlockSpec((BLOCK_T, EMBEDDING_DIM), lambda i: (i, 0)),
            pl.BlockSpec((CODE_DIM, NUM_EMBEDDINGS), lambda i: (0, 0)),
            pl.BlockSpec((NUM_EMBEDDINGS, CODE_DIM), lambda i: (0, 0)),
            pl.BlockSpec((1, NUM_EMBEDDINGS), lambda i: (0, 0)),
        ],
        out_specs=[
            pl.BlockSpec((BLOCK_T, EMBEDDING_DIM), lambda i: (i, 0)),
            pl.BlockSpec((1, 1), lambda i: (0, 0)),
            pl.BlockSpec((1, 1), lambda i: (0, 0)),
        ],
        out_shape=[
            jax.ShapeDtypeStruct((n_tok, EMBEDDING_DIM), jnp.float32),
            jax.ShapeDtypeStruct((1, 1), jnp.float32),
            jax.ShapeDtypeStruct((1, 1), jnp.float32),
        ],
        scratch_shapes=[
            pltpu.SMEM((1, 1), jnp.float32),
            pltpu.VMEM((1, NUM_EMBEDDINGS), jnp.float32),
        ],
    )(x2, wt, W, wsq)

    return (q.reshape(input_shape), loss[0, 0], perp[0, 0])


# speedup vs baseline: 5.7277x; 1.0949x over previous
import functools

import jax
import jax.numpy as jnp
from jax import lax
from jax.experimental import pallas as pl
from jax.experimental.pallas import tpu as pltpu
from jax.experimental.pallas import tpu_sc as plsc

NUM_EMBEDDINGS = 1024
EMBEDDING_DIM = 256
SAMPLE_TOKENS = 8
CODE_DIM = 32
COMMITMENT_COST = 0.25

BLOCK_T = 512

_SC_INFO = plsc.get_sparse_core_info()
_NC, _NS, _L = _SC_INFO.num_cores, _SC_INFO.num_subcores, _SC_INFO.num_lanes
_NW = _NC * _NS


def _tc_body(x_ref, wt_ref, wsq_ref, idx_ref, loss_ref, acc_ref,
             *, n_rows, n_blocks):
    step = pl.program_id(0)
    wt = wt_ref[...]
    wsq = wsq_ref[...]

    blk_sq = None
    for s in range(SAMPLE_TOKENS):
        xs = x_ref[:, s * CODE_DIM:(s + 1) * CODE_DIM]
        scores = jnp.dot(xs, wt, preferred_element_type=jnp.float32)
        xsq = jnp.sum(xs * xs, axis=1, keepdims=True)
        dist = (xsq + wsq) - 2.0 * scores
        dmin = jnp.min(dist, axis=1, keepdims=True)
        col = jax.lax.broadcasted_iota(jnp.int32, dist.shape, 1)
        idx = jnp.min(jnp.where(dist == dmin, col, NUM_EMBEDDINGS),
                      axis=1, keepdims=True)
        idx_ref[:, s:s + 1] = idx
        sq_s = jnp.sum(dmin)
        blk_sq = sq_s if blk_sq is None else blk_sq + sq_s

    @pl.when(step == 0)
    def _init():
        acc_ref[0, 0] = blk_sq

    @pl.when(step > 0)
    def _acc():
        acc_ref[0, 0] += blk_sq

    @pl.when(step == n_blocks - 1)
    def _finalize():
        total_sq = acc_ref[0, 0]
        loss = (1.0 + COMMITMENT_COST) * total_sq / (n_rows * CODE_DIM)
        loss_ref[...] = jnp.reshape(loss, (1, 1))


def _make_sc_kernel(n_rows):
    bpw = n_rows // _NW
    mesh = plsc.VectorSubcoreMesh(core_axis_name="c", subcore_axis_name="s")

    @functools.partial(
        pl.kernel, mesh=mesh,
        out_type=[
            jax.ShapeDtypeStruct((n_rows, CODE_DIM), jnp.float32),
            jax.ShapeDtypeStruct((_NC, NUM_EMBEDDINGS), jnp.float32),
        ],
        scratch_types=[
            pltpu.VMEM((bpw,), jnp.int32),
            pltpu.VMEM((bpw, CODE_DIM), jnp.float32),
            pltpu.VMEM((bpw,), jnp.float32),
            pltpu.VMEM((NUM_EMBEDDINGS,), jnp.float32),
            pltpu.VMEM_SHARED((NUM_EMBEDDINGS,), jnp.float32),
            pltpu.SemaphoreType.DMA,
        ],
        compiler_params=pltpu.CompilerParams(use_tc_tiling_on_sc=False),
    )
    def sc_gather_hist(table_hbm, idx_hbm, out_hbm, cnt_hbm, idx_v, rows_v,
                       ones_v, bounce_v, cnt_sh, sem):
        cid = lax.axis_index("c")
        sid = lax.axis_index("s")
        wid = sid * _NC + cid
        base = wid * bpw
        pltpu.sync_copy(idx_hbm.at[pl.ds(base, bpw)], idx_v)
        pltpu.async_copy(table_hbm.at[idx_v], rows_v, sem).wait()
        pltpu.sync_copy(rows_v, out_hbm.at[pl.ds(base, bpw)])

        zero = jnp.zeros((_L,), jnp.float32)
        one = jnp.ones((_L,), jnp.float32)
        for i in range(bpw // _L):
            ones_v[pl.ds(i * _L, _L)] = one

        @pl.when(sid == 0)
        def _init():
            for i in range(NUM_EMBEDDINGS // _L):
                bounce_v[pl.ds(i * _L, _L)] = zero
            pltpu.sync_copy(bounce_v, cnt_sh)

        plsc.subcore_barrier()
        pltpu.sync_copy(ones_v, cnt_sh.at[idx_v], add=True)
        plsc.subcore_barrier()

        @pl.when(sid == 0)
        def _emit():
            pltpu.sync_copy(cnt_sh, bounce_v)
            pltpu.sync_copy(bounce_v, cnt_hbm.at[cid])

    return sc_gather_hist


def _perp_body(cnt_ref, perp_ref, *, n_rows):
    c = cnt_ref[...]
    probs = jnp.sum(c, axis=0, keepdims=True) / n_rows
    ent = jnp.sum(probs * jnp.log(probs + 1e-10), axis=1, keepdims=True)
    perp_ref[...] = jnp.exp(-ent)


def kernel(inputs, W):
    input_shape = inputs.shape
    x2 = inputs.reshape(-1, EMBEDDING_DIM)
    n_tok = x2.shape[0]
    n_rows = n_tok * SAMPLE_TOKENS
    n_blocks = n_tok // BLOCK_T

    wt = W.T
    wsq = jnp.sum(W ** 2, axis=1)[None, :]
    table = W.astype(jnp.bfloat16).astype(jnp.float32)

    idx_aos, loss = pl.pallas_call(
        functools.partial(_tc_body, n_rows=n_rows, n_blocks=n_blocks),
        grid=(n_blocks,),
        in_specs=[
            pl.BlockSpec((BLOCK_T, EMBEDDING_DIM), lambda i: (i, 0)),
            pl.BlockSpec((CODE_DIM, NUM_EMBEDDINGS), lambda i: (0, 0)),
            pl.BlockSpec((1, NUM_EMBEDDINGS), lambda i: (0, 0)),
        ],
        out_specs=[
            pl.BlockSpec((BLOCK_T, SAMPLE_TOKENS), lambda i: (i, 0)),
            pl.BlockSpec((1, 1), lambda i: (0, 0)),
        ],
        out_shape=[
            jax.ShapeDtypeStruct((n_tok, SAMPLE_TOKENS), jnp.int32),
            jax.ShapeDtypeStruct((1, 1), jnp.float32),
        ],
        scratch_shapes=[
            pltpu.SMEM((1, 1), jnp.float32),
        ],
    )(x2, wt, wsq)

    idx_flat = idx_aos.reshape(-1)

    q_flat, cnt = _make_sc_kernel(n_rows)(table, idx_flat)

    perp = pl.pallas_call(
        functools.partial(_perp_body, n_rows=n_rows),
        in_specs=[pl.BlockSpec((_NC, NUM_EMBEDDINGS), lambda: (0, 0))],
        out_specs=pl.BlockSpec((1, 1), lambda: (0, 0)),
        out_shape=jax.ShapeDtypeStruct((1, 1), jnp.float32),
    )(cnt)

    return (q_flat.reshape(input_shape), loss[0, 0], perp[0, 0])
